# baseline (speedup 1.0000x reference)
import functools
import jax
import jax.numpy as jnp
from jax import lax
from jax.experimental import pallas as pl
from jax.experimental.pallas import tpu as pltpu
from jax.experimental.pallas import tpu_sc as plsc

_N = 20000
_NPAD = 20480
_NTILES = 16
_CHUNK = _NPAD // _NTILES
_VREGS = _CHUNK // 16
_MAX_DET = 300
_NEG = -1e30
_BIG = 2**30
_GUARD = 16


def _argmax_butterfly(lane, val, idx, bf, bi_ref):
    for d in (8, 4, 2, 1):
        perm = lane ^ d
        bf[...] = val
        bi_ref[...] = idx
        vo = plsc.load_gather(bf, [perm])
        io = plsc.load_gather(bi_ref, [perm])
        better = (vo > val) | ((vo == val) & (io < idx))
        val = jnp.where(better, vo, val)
        idx = jnp.where(better, io, idx)
    return val, idx


def _sc_body(coords_hbm, scores_hbm, out_hbm,
             x1v, y1v, x2v, y2v, sv, wk, ar, recv, allrec, rows, bf, bi_b, shm):
    wid = lax.axis_index("s")
    base = wid * _CHUNK
    lane = lax.broadcasted_iota(jnp.int32, (16,), 0)

    pltpu.sync_copy(coords_hbm.at[0, pl.ds(base, _CHUNK)], x1v)
    pltpu.sync_copy(coords_hbm.at[1, pl.ds(base, _CHUNK)], y1v)
    pltpu.sync_copy(coords_hbm.at[2, pl.ds(base, _CHUNK)], x2v)
    pltpu.sync_copy(coords_hbm.at[3, pl.ds(base, _CHUNK)], y2v)
    pltpu.sync_copy(scores_hbm.at[pl.ds(base, _CHUNK)], sv)

    for j in range(_VREGS):
        sl = pl.ds(j * 16, 16)
        s = sv[sl]
        wk[sl] = jnp.where(s > 0.05, s, _NEG)
        ar[sl] = (x2v[sl] - x1v[sl]) * (y2v[sl] - y1v[sl])

    def step(t, carry):
        bv = wk[pl.ds(0, 16)]
        bi = lane
        for j in range(1, _VREGS):
            v = wk[pl.ds(j * 16, 16)]
            gt = v > bv
            bv = jnp.where(gt, v, bv)
            bi = jnp.where(gt, j * 16 + lane, bi)
        m, li = _argmax_butterfly(lane, bv, bi, bf, bi_b)
        gidxv = base + li

        bx1l = plsc.load_gather(x1v, [li])
        by1l = plsc.load_gather(y1v, [li])
        bx2l = plsc.load_gather(x2v, [li])
        by2l = plsc.load_gather(y2v, [li])

        gvec = plsc.bitcast(gidxv, jnp.float32)
        rec = jnp.where(
            lane == 0, m,
            jnp.where(lane == 1, gvec,
                      jnp.where(lane == 2, bx1l,
                                jnp.where(lane == 3, by1l,
                                          jnp.where(lane == 4, bx2l, by2l)))))
        recv[...] = rec
        pltpu.sync_copy(recv, shm.at[_GUARD + wid])
        plsc.subcore_barrier()
        pltpu.sync_copy(shm.at[pl.ds(_GUARD, _NTILES)], allrec)
        plsc.subcore_barrier()

        zero = jnp.zeros((16,), jnp.int32)
        mv = plsc.load_gather(allrec, [lane, zero])
        gv = plsc.bitcast(plsc.load_gather(allrec, [lane, zero + 1]), jnp.int32)
        M, G = _argmax_butterfly(lane, mv, gv, bf, bi_b)
        wtv = lax.div(G, jnp.int32(_CHUNK))
        bx1 = plsc.load_gather(allrec, [wtv, zero + 2])
        by1 = plsc.load_gather(allrec, [wtv, zero + 3])
        bx2 = plsc.load_gather(allrec, [wtv, zero + 4])
        by2 = plsc.load_gather(allrec, [wtv, zero + 5])
        a1 = (bx2 - bx1) * (by2 - by1)

        for j in range(_VREGS):
            sl = pl.ds(j * 16, 16)
            xx1 = x1v[sl]
            yy1 = y1v[sl]
            xx2 = x2v[sl]
            yy2 = y2v[sl]
            ix1 = jnp.maximum(bx1, xx1)
            iy1 = jnp.maximum(by1, yy1)
            ix2 = jnp.minimum(bx2, xx2)
            iy2 = jnp.minimum(by2, yy2)
            inter = jnp.maximum(ix2 - ix1, 0.0) * jnp.maximum(iy2 - iy1, 0.0)
            iou = inter / (a1 + ar[sl] - inter + 1e-6)
            issel = (base + j * 16 + lane) == G
            wk[sl] = jnp.where((iou > 0.5) | issel, _NEG, wk[sl])

        valid = M > 0.05
        vf = jnp.where(valid, 1.0, 0.0)
        sc = jnp.where(valid, M, 0.0)
        row = jnp.where(
            lane == 0, bx1 * vf,
            jnp.where(lane == 1, by1 * vf,
                      jnp.where(lane == 2, bx2 * vf,
                                jnp.where(lane == 3, by2 * vf,
                                          jnp.where(lane == 4, sc, 0.0)))))
        plsc.store_scatter(rows, [t * 8 + lane], row)

        return carry

    lax.fori_loop(0, _MAX_DET, step, jnp.int32(0))

    @pl.when(wid == 0)
    def _():
        pltpu.sync_copy(rows.at[pl.ds(0, _MAX_DET * 8)], out_hbm)


def kernel(boxes, scores):
    x1 = boxes[:, 0] * 800.0
    y1 = boxes[:, 1] * 800.0
    x2 = x1 + boxes[:, 2] * 96.0 + 1.0
    y2 = y1 + boxes[:, 3] * 96.0 + 1.0
    coords = jnp.stack([x1, y1, x2, y2], axis=0)
    coords = jnp.pad(coords, ((0, 0), (0, _NPAD - _N)))
    s = jnp.pad(scores, (0, _NPAD - _N))

    mesh = plsc.VectorSubcoreMesh(
        core_axis_name="c", subcore_axis_name="s", num_cores=1,
        num_subcores=_NTILES)
    out = pl.kernel(
        _sc_body,
        out_type=jax.ShapeDtypeStruct((_MAX_DET * 8,), jnp.float32),
        mesh=mesh,
        compiler_params=pltpu.CompilerParams(needs_layout_passes=False),
        scratch_types=[
            pltpu.VMEM((_CHUNK,), jnp.float32),
            pltpu.VMEM((_CHUNK,), jnp.float32),
            pltpu.VMEM((_CHUNK,), jnp.float32),
            pltpu.VMEM((_CHUNK,), jnp.float32),
            pltpu.VMEM((_CHUNK,), jnp.float32),
            pltpu.VMEM((_CHUNK,), jnp.float32),
            pltpu.VMEM((_CHUNK,), jnp.float32),
            pltpu.VMEM((16,), jnp.float32),
            pltpu.VMEM((_NTILES, 16), jnp.float32),
            pltpu.VMEM((_MAX_DET * 8 + 16,), jnp.float32),
            pltpu.VMEM((16,), jnp.float32),
            pltpu.VMEM((16,), jnp.int32),
            pltpu.VMEM_SHARED((_GUARD + _NTILES, 16), jnp.float32),
        ],
    )(coords, s)
    return out.reshape(_MAX_DET, 8)[:, :5]


# speedup vs baseline: 1.1812x; 1.1812x over previous
import jax
import jax.numpy as jnp
from jax import lax
from jax.experimental import pallas as pl
from jax.experimental.pallas import tpu as pltpu
from jax.experimental.pallas import tpu_sc as plsc

_N = 20000
_NPAD = 20480
_NTILES = 16
_CHUNK = _NPAD // _NTILES
_VREGS = _CHUNK // 16
_MAX_DET = 300
_NEG = -1e30
_GUARD = 16


def _argmax_butterfly(lane, val, idx, bf, bi_ref):
    for d in (8, 4, 2, 1):
        perm = lane ^ d
        bf[...] = val
        bi_ref[...] = idx
        vo = plsc.load_gather(bf, [perm])
        io = plsc.load_gather(bi_ref, [perm])
        better = (vo > val) | ((vo == val) & (io < idx))
        val = jnp.where(better, vo, val)
        idx = jnp.where(better, io, idx)
    return val, idx


def _sc_body(coords_hbm, scores_hbm, out_hbm,
             x1v, y1v, x2v, y2v, sv, wk, ar, recv, allrec, rows, bf, bi_b, shm):
    wid = lax.axis_index("s")
    base = wid * _CHUNK
    lane = lax.broadcasted_iota(jnp.int32, (16,), 0)

    pltpu.sync_copy(coords_hbm.at[0, pl.ds(base, _CHUNK)], x1v)
    pltpu.sync_copy(coords_hbm.at[1, pl.ds(base, _CHUNK)], y1v)
    pltpu.sync_copy(coords_hbm.at[2, pl.ds(base, _CHUNK)], x2v)
    pltpu.sync_copy(coords_hbm.at[3, pl.ds(base, _CHUNK)], y2v)
    pltpu.sync_copy(scores_hbm.at[pl.ds(base, _CHUNK)], sv)

    for j in range(_VREGS):
        sl = pl.ds(j * 16, 16)
        s = sv[sl]
        wk[sl] = jnp.where(s > 0.05, s, _NEG)
        ar[sl] = (x2v[sl] - x1v[sl]) * (y2v[sl] - y1v[sl])

    def step(t, carry):
        bv, bi = carry
        m, li = _argmax_butterfly(lane, bv, bi, bf, bi_b)
        pk = (wid * 2048) + li

        bx1l = plsc.load_gather(x1v, [li])
        by1l = plsc.load_gather(y1v, [li])
        bx2l = plsc.load_gather(x2v, [li])
        by2l = plsc.load_gather(y2v, [li])

        gvec = plsc.bitcast(pk, jnp.float32)
        a1l = (bx2l - bx1l) * (by2l - by1l)
        rec = jnp.where(
            lane == 0, m,
            jnp.where(lane == 1, gvec,
                      jnp.where(lane == 2, bx1l,
                                jnp.where(lane == 3, by1l,
                                          jnp.where(lane == 4, bx2l,
                                                    jnp.where(lane == 5, by2l, a1l))))))
        recv[...] = rec
        po = _GUARD + (t & 1) * _NTILES
        pltpu.sync_copy(recv, shm.at[po + wid])
        plsc.subcore_barrier()
        pltpu.sync_copy(shm.at[pl.ds(po, _NTILES)], allrec)

        zero = jnp.zeros((16,), jnp.int32)
        mv = plsc.load_gather(allrec, [lane, zero])
        gv = plsc.bitcast(plsc.load_gather(allrec, [lane, zero + 1]), jnp.int32)
        M, G = _argmax_butterfly(lane, mv, gv, bf, bi_b)
        wtv = lax.shift_right_logical(G, 11)
        bx1 = plsc.load_gather(allrec, [wtv, zero + 2])
        by1 = plsc.load_gather(allrec, [wtv, zero + 3])
        bx2 = plsc.load_gather(allrec, [wtv, zero + 4])
        by2 = plsc.load_gather(allrec, [wtv, zero + 5])
        a1 = plsc.load_gather(allrec, [wtv, zero + 6])

        nbv = jnp.full((16,), _NEG, dtype=jnp.float32)
        nbi = jnp.zeros((16,), jnp.int32)
        for j in range(_VREGS):
            sl = pl.ds(j * 16, 16)
            xx1 = x1v[sl]
            yy1 = y1v[sl]
            xx2 = x2v[sl]
            yy2 = y2v[sl]
            ix1 = jnp.maximum(bx1, xx1)
            iy1 = jnp.maximum(by1, yy1)
            ix2 = jnp.minimum(bx2, xx2)
            iy2 = jnp.minimum(by2, yy2)
            inter = jnp.maximum(ix2 - ix1, 0.0) * jnp.maximum(iy2 - iy1, 0.0)
            iou = inter / (a1 + ar[sl] - inter + 1e-6)
            nw = jnp.where(iou > 0.5, _NEG, wk[sl])
            wk[sl] = nw
            gt = nw > nbv
            nbv = jnp.where(gt, nw, nbv)
            nbi = jnp.where(gt, j * 16 + lane, nbi)

        valid = M > 0.05
        vf = jnp.where(valid, 1.0, 0.0)
        sc = jnp.where(valid, M, 0.0)
        row = jnp.where(
            lane == 0, bx1 * vf,
            jnp.where(lane == 1, by1 * vf,
                      jnp.where(lane == 2, bx2 * vf,
                                jnp.where(lane == 3, by2 * vf,
                                          jnp.where(lane == 4, sc, 0.0)))))
        plsc.store_scatter(rows, [t * 8 + lane], row)

        return (nbv, nbi)

    bv0 = jnp.full((16,), _NEG, dtype=jnp.float32)
    bi0 = jnp.zeros((16,), jnp.int32)
    for j in range(_VREGS):
        w = wk[pl.ds(j * 16, 16)]
        gt = w > bv0
        bv0 = jnp.where(gt, w, bv0)
        bi0 = jnp.where(gt, j * 16 + lane, bi0)
    lax.fori_loop(0, _MAX_DET, step, (bv0, bi0))

    @pl.when(wid == 0)
    def _():
        pltpu.sync_copy(rows.at[pl.ds(0, _MAX_DET * 8)], out_hbm)


def kernel(boxes, scores):
    x1 = boxes[:, 0] * 800.0
    y1 = boxes[:, 1] * 800.0
    x2 = x1 + boxes[:, 2] * 96.0 + 1.0
    y2 = y1 + boxes[:, 3] * 96.0 + 1.0
    coords = jnp.stack([x1, y1, x2, y2], axis=0)
    coords = jnp.pad(coords, ((0, 0), (0, _NPAD - _N)))
    s = jnp.pad(scores, (0, _NPAD - _N))

    mesh = plsc.VectorSubcoreMesh(
        core_axis_name="c", subcore_axis_name="s", num_cores=1,
        num_subcores=_NTILES)
    out = pl.kernel(
        _sc_body,
        out_type=jax.ShapeDtypeStruct((_MAX_DET * 8,), jnp.float32),
        mesh=mesh,
        compiler_params=pltpu.CompilerParams(needs_layout_passes=False),
        scratch_types=[
            pltpu.VMEM((_CHUNK,), jnp.float32),
            pltpu.VMEM((_CHUNK,), jnp.float32),
            pltpu.VMEM((_CHUNK,), jnp.float32),
            pltpu.VMEM((_CHUNK,), jnp.float32),
            pltpu.VMEM((_CHUNK,), jnp.float32),
            pltpu.VMEM((_CHUNK,), jnp.float32),
            pltpu.VMEM((_CHUNK,), jnp.float32),
            pltpu.VMEM((16,), jnp.float32),
            pltpu.VMEM((_NTILES, 16), jnp.float32),
            pltpu.VMEM((_MAX_DET * 8 + 16,), jnp.float32),
            pltpu.VMEM((16,), jnp.float32),
            pltpu.VMEM((16,), jnp.int32),
            pltpu.VMEM_SHARED((_GUARD + 2 * _NTILES, 16), jnp.float32),
        ],
    )(coords, s)
    return out.reshape(_MAX_DET, 8)[:, :5]
